# Initial kernel scaffold; baseline (speedup 1.0000x reference)
import jax
import jax.numpy as jnp
from jax.experimental import pallas as pl


def kernel(Q, K, V, mask):
    raise NotImplementedError("write your pallas kernel here")


# speedup vs baseline: 1.2041x; 1.2041x over previous
import functools
import math

import jax
import jax.numpy as jnp
from jax.experimental import pallas as pl

_INV_PI = 1.0 / math.pi
_PI = math.pi

_ACOS_COEFFS = (
    -0.0012624911,
    0.0066700901,
    -0.0170881256,
    0.0308918810,
    -0.0501743046,
    0.0889789874,
    -0.2145988016,
    1.5707963050,
)


def _acos(x):
    ax = jnp.abs(x)
    p = jnp.float32(_ACOS_COEFFS[0])
    for c in _ACOS_COEFFS[1:]:
        p = p * ax + jnp.float32(c)
    r = jnp.sqrt(1.0 - ax) * p
    return jnp.where(x >= 0, r, jnp.float32(_PI) - r)


def _row_normalize(x):
    n = jnp.sqrt(jnp.sum(x * x, axis=-1, keepdims=True))
    return x / jnp.clip(n, 1e-12, None)


def _yoso_block_kernel(q_ref, k_ref, v_ref, m_ref, o_ref, *, bq):
    iq = pl.program_id(1)
    q = q_ref[0]
    k = k_ref[0]
    v = v_ref[0]
    m = m_ref[0]

    qn = _row_normalize(q)
    kn = _row_normalize(k)

    s = jax.lax.dot_general(
        qn, kn, (((1,), (1,)), ((), ())), preferred_element_type=jnp.float32
    )
    s = jnp.clip(s, -0.99999, 0.99999)
    t = 1.0 - _acos(s) * jnp.float32(_INV_PI)
    t2 = t * t
    t4 = t2 * t2
    t8 = t4 * t4
    e = t8 * t
    e = e * m[None, :]

    x = jax.lax.dot_general(
        e, v, (((1,), (0,)), ((), ())), preferred_element_type=jnp.float32
    )
    mq = m_ref[0, pl.ds(iq * bq, bq)]
    x = x * mq[:, None]
    o_ref[0] = _row_normalize(x)


def kernel(Q, K, V, mask):
    B, H, S, D = Q.shape
    BH = B * H
    q = Q.reshape(BH, S, D)
    k = K.reshape(BH, S, D)
    v = V.reshape(BH, S, D)
    mf = mask.astype(jnp.float32)

    bq = min(256, S)
    grid = (BH, S // bq)

    out = pl.pallas_call(
        functools.partial(_yoso_block_kernel, bq=bq),
        grid=grid,
        in_specs=[
            pl.BlockSpec((1, bq, D), lambda h, i: (h, i, 0)),
            pl.BlockSpec((1, S, D), lambda h, i: (h, 0, 0)),
            pl.BlockSpec((1, S, D), lambda h, i: (h, 0, 0)),
            pl.BlockSpec((1, S), lambda h, i: (0, 0)),
        ],
        out_specs=pl.BlockSpec((1, bq, D), lambda h, i: (h, i, 0)),
        out_shape=jax.ShapeDtypeStruct((BH, S, D), jnp.float32),
    )(q, k, v, mf)

    return out.reshape(B, H, S, D)


# speedup vs baseline: 1.5840x; 1.3155x over previous
import functools
import math

import jax
import jax.numpy as jnp
from jax.experimental import pallas as pl

_ACOS_PI_COEFFS = (
    -0.0187293 / math.pi,
    0.0742610 / math.pi,
    -0.2121144 / math.pi,
    1.5707288 / math.pi,
)


def _row_normalize(x):
    n = jnp.sqrt(jnp.sum(x * x, axis=-1, keepdims=True))
    return x / jnp.clip(n, 1e-12, None)


def _prep_kernel(q_ref, k_ref, v_ref, m_ref, qo_ref, ko_ref, vo_ref):
    m = m_ref[0]
    qo_ref[0] = _row_normalize(q_ref[0]).astype(jnp.bfloat16)
    ko_ref[0] = _row_normalize(k_ref[0]).astype(jnp.bfloat16)
    vo_ref[0] = (v_ref[0] * m[:, None]).astype(jnp.bfloat16)


def _attn_kernel(q_ref, k_ref, v_ref, m_ref, o_ref, *, bq):
    iq = pl.program_id(1)
    s = jax.lax.dot_general(
        q_ref[0], k_ref[0], (((1,), (1,)), ((), ())),
        preferred_element_type=jnp.float32,
    )
    ax = jnp.minimum(jnp.abs(s), 0.99999)
    p = jnp.float32(_ACOS_PI_COEFFS[0])
    for c in _ACOS_PI_COEFFS[1:]:
        p = p * ax + jnp.float32(c)
    r = jnp.sqrt(1.0 - ax) * p
    t = jnp.where(s >= 0, 1.0 - r, r)
    t2 = t * t
    t4 = t2 * t2
    t8 = t4 * t4
    e = (t8 * t).astype(jnp.bfloat16)
    x = jax.lax.dot_general(
        e, v_ref[0], (((1,), (0,)), ((), ())),
        preferred_element_type=jnp.float32,
    )
    mq = m_ref[0, pl.ds(iq * bq, bq)]
    o_ref[0] = _row_normalize(x * mq[:, None])


def kernel(Q, K, V, mask):
    B, H, S, D = Q.shape
    BH = B * H
    q = Q.reshape(BH, S, D)
    k = K.reshape(BH, S, D)
    v = V.reshape(BH, S, D)
    mf = mask.astype(jnp.float32)

    qn, kn, vm = pl.pallas_call(
        _prep_kernel,
        grid=(BH,),
        in_specs=[
            pl.BlockSpec((1, S, D), lambda h: (h, 0, 0)),
            pl.BlockSpec((1, S, D), lambda h: (h, 0, 0)),
            pl.BlockSpec((1, S, D), lambda h: (h, 0, 0)),
            pl.BlockSpec((1, S), lambda h: (0, 0)),
        ],
        out_specs=[
            pl.BlockSpec((1, S, D), lambda h: (h, 0, 0)),
            pl.BlockSpec((1, S, D), lambda h: (h, 0, 0)),
            pl.BlockSpec((1, S, D), lambda h: (h, 0, 0)),
        ],
        out_shape=[
            jax.ShapeDtypeStruct((BH, S, D), jnp.bfloat16),
            jax.ShapeDtypeStruct((BH, S, D), jnp.bfloat16),
            jax.ShapeDtypeStruct((BH, S, D), jnp.bfloat16),
        ],
    )(q, k, v, mf)

    bq = min(256, S)
    grid = (BH, S // bq)

    out = pl.pallas_call(
        functools.partial(_attn_kernel, bq=bq),
        grid=grid,
        in_specs=[
            pl.BlockSpec((1, bq, D), lambda h, i: (h, i, 0)),
            pl.BlockSpec((1, S, D), lambda h, i: (h, 0, 0)),
            pl.BlockSpec((1, S, D), lambda h, i: (h, 0, 0)),
            pl.BlockSpec((1, S), lambda h, i: (0, 0)),
        ],
        out_specs=pl.BlockSpec((1, bq, D), lambda h, i: (h, i, 0)),
        out_shape=jax.ShapeDtypeStruct((BH, S, D), jnp.float32),
    )(qn, kn, vm, mf)

    return out.reshape(B, H, S, D)


# speedup vs baseline: 1.7616x; 1.1121x over previous
import functools
import math

import jax
import jax.numpy as jnp
from jax.experimental import pallas as pl

_ACOS_PI_COEFFS = (
    -0.0187293 / math.pi,
    0.0742610 / math.pi,
    -0.2121144 / math.pi,
    1.5707288 / math.pi,
)


def _row_normalize(x):
    n2 = jnp.sum(x * x, axis=-1, keepdims=True)
    return x * jax.lax.rsqrt(jnp.maximum(n2, 1e-24))


def _prep_kernel(q_ref, k_ref, v_ref, m_ref, qo_ref, ko_ref, vo_ref):
    m = m_ref[0]
    qo_ref[0] = _row_normalize(q_ref[0]).astype(jnp.bfloat16)
    ko_ref[0] = _row_normalize(k_ref[0]).astype(jnp.bfloat16)
    vo_ref[0] = (v_ref[0] * m[:, None]).astype(jnp.bfloat16)


def _attn_kernel(q_ref, k_ref, v_ref, m_ref, o_ref, *, bq):
    iq = pl.program_id(1)
    s = jax.lax.dot_general(
        q_ref[0], k_ref[0], (((1,), (1,)), ((), ())),
        preferred_element_type=jnp.float32,
    )
    ax = jnp.minimum(jnp.abs(s), 0.99999)
    p = jnp.float32(_ACOS_PI_COEFFS[0])
    for c in _ACOS_PI_COEFFS[1:]:
        p = p * ax + jnp.float32(c)
    y = 1.0 - ax
    r = (y * jax.lax.rsqrt(y)) * p
    t = jnp.where(s >= 0, 1.0 - r, r)
    t2 = t * t
    t4 = t2 * t2
    t8 = t4 * t4
    e = (t8 * t).astype(jnp.bfloat16)
    x = jax.lax.dot_general(
        e, v_ref[0], (((1,), (0,)), ((), ())),
        preferred_element_type=jnp.float32,
    )
    mq = m_ref[0, pl.ds(iq * bq, bq)]
    o_ref[0] = _row_normalize(x * mq[:, None])


def kernel(Q, K, V, mask):
    B, H, S, D = Q.shape
    BH = B * H
    q = Q.reshape(BH, S, D)
    k = K.reshape(BH, S, D)
    v = V.reshape(BH, S, D)
    mf = mask.astype(jnp.float32)

    qn, kn, vm = pl.pallas_call(
        _prep_kernel,
        grid=(BH,),
        in_specs=[
            pl.BlockSpec((1, S, D), lambda h: (h, 0, 0)),
            pl.BlockSpec((1, S, D), lambda h: (h, 0, 0)),
            pl.BlockSpec((1, S, D), lambda h: (h, 0, 0)),
            pl.BlockSpec((1, S), lambda h: (0, 0)),
        ],
        out_specs=[
            pl.BlockSpec((1, S, D), lambda h: (h, 0, 0)),
            pl.BlockSpec((1, S, D), lambda h: (h, 0, 0)),
            pl.BlockSpec((1, S, D), lambda h: (h, 0, 0)),
        ],
        out_shape=[
            jax.ShapeDtypeStruct((BH, S, D), jnp.bfloat16),
            jax.ShapeDtypeStruct((BH, S, D), jnp.bfloat16),
            jax.ShapeDtypeStruct((BH, S, D), jnp.bfloat16),
        ],
    )(q, k, v, mf)

    bq = min(256, S)
    grid = (BH, S // bq)

    out = pl.pallas_call(
        functools.partial(_attn_kernel, bq=bq),
        grid=grid,
        in_specs=[
            pl.BlockSpec((1, bq, D), lambda h, i: (h, i, 0)),
            pl.BlockSpec((1, S, D), lambda h, i: (h, 0, 0)),
            pl.BlockSpec((1, S, D), lambda h, i: (h, 0, 0)),
            pl.BlockSpec((1, S), lambda h, i: (0, 0)),
        ],
        out_specs=pl.BlockSpec((1, bq, D), lambda h, i: (h, i, 0)),
        out_shape=jax.ShapeDtypeStruct((BH, S, D), jnp.float32),
    )(qn, kn, vm, mf)

    return out.reshape(B, H, S, D)


# speedup vs baseline: 1.9705x; 1.1186x over previous
import functools
import math

import jax
import jax.numpy as jnp
from jax.experimental import pallas as pl

_ACOS_PI_COEFFS = (
    -0.0187293 / math.pi,
    0.0742610 / math.pi,
    -0.2121144 / math.pi,
    1.5707288 / math.pi,
)


def _row_normalize(x):
    n2 = jnp.sum(x * x, axis=-1, keepdims=True)
    return x * jax.lax.rsqrt(jnp.maximum(n2, 1e-24))


def _prep_kernel(q_ref, k_ref, v_ref, m_ref, qo_ref, ko_ref, vo_ref):
    m = m_ref[0]
    qo_ref[0] = _row_normalize(q_ref[0]).astype(jnp.bfloat16)
    ko_ref[0] = _row_normalize(k_ref[0]).astype(jnp.bfloat16)
    vo_ref[0] = (v_ref[0] * m[:, None]).astype(jnp.bfloat16)


def _attn_kernel(q_ref, k_ref, v_ref, m_ref, o_ref, *, bq):
    iq = pl.program_id(1)
    s = jax.lax.dot_general(
        q_ref[0], k_ref[0], (((1,), (1,)), ((), ())),
        preferred_element_type=jnp.float32,
    )
    ax = jnp.minimum(jnp.abs(s), 0.99999)
    p = jnp.float32(_ACOS_PI_COEFFS[0])
    for c in _ACOS_PI_COEFFS[1:]:
        p = p * ax + jnp.float32(c)
    y = 1.0 - ax
    r = (y * jax.lax.rsqrt(y)) * p
    t = jnp.where(s >= 0, 1.0 - r, r)
    t2 = t * t
    t4 = t2 * t2
    t8 = t4 * t4
    e = (t8 * t).astype(jnp.bfloat16)
    x = jax.lax.dot_general(
        e, v_ref[0], (((1,), (0,)), ((), ())),
        preferred_element_type=jnp.float32,
    )
    mq = m_ref[0, pl.ds(iq * bq, bq)]
    o_ref[0] = _row_normalize(x * mq[:, None])


def kernel(Q, K, V, mask):
    B, H, S, D = Q.shape
    BH = B * H
    q = Q.reshape(BH, S, D)
    k = K.reshape(BH, S, D)
    v = V.reshape(BH, S, D)
    mf = mask.astype(jnp.float32)

    qn, kn, vm = pl.pallas_call(
        _prep_kernel,
        grid=(BH,),
        in_specs=[
            pl.BlockSpec((1, S, D), lambda h: (h, 0, 0)),
            pl.BlockSpec((1, S, D), lambda h: (h, 0, 0)),
            pl.BlockSpec((1, S, D), lambda h: (h, 0, 0)),
            pl.BlockSpec((1, S), lambda h: (0, 0)),
        ],
        out_specs=[
            pl.BlockSpec((1, S, D), lambda h: (h, 0, 0)),
            pl.BlockSpec((1, S, D), lambda h: (h, 0, 0)),
            pl.BlockSpec((1, S, D), lambda h: (h, 0, 0)),
        ],
        out_shape=[
            jax.ShapeDtypeStruct((BH, S, D), jnp.bfloat16),
            jax.ShapeDtypeStruct((BH, S, D), jnp.bfloat16),
            jax.ShapeDtypeStruct((BH, S, D), jnp.bfloat16),
        ],
    )(q, k, v, mf)

    bq = min(512, S)
    grid = (BH, S // bq)

    out = pl.pallas_call(
        functools.partial(_attn_kernel, bq=bq),
        grid=grid,
        in_specs=[
            pl.BlockSpec((1, bq, D), lambda h, i: (h, i, 0)),
            pl.BlockSpec((1, S, D), lambda h, i: (h, 0, 0)),
            pl.BlockSpec((1, S, D), lambda h, i: (h, 0, 0)),
            pl.BlockSpec((1, S), lambda h, i: (0, 0)),
        ],
        out_specs=pl.BlockSpec((1, bq, D), lambda h, i: (h, i, 0)),
        out_shape=jax.ShapeDtypeStruct((BH, S, D), jnp.float32),
    )(qn, kn, vm, mf)

    return out.reshape(B, H, S, D)


# speedup vs baseline: 2.1080x; 1.0698x over previous
import functools
import math

import jax
import jax.numpy as jnp
from jax.experimental import pallas as pl

_ACOS_PI_COEFFS = (
    -0.0187293 / math.pi,
    0.0742610 / math.pi,
    -0.2121144 / math.pi,
    1.5707288 / math.pi,
)


def _row_normalize(x):
    n2 = jnp.sum(x * x, axis=-1, keepdims=True)
    return x * jax.lax.rsqrt(jnp.maximum(n2, 1e-24))


def _prep_kernel(q_ref, k_ref, v_ref, m_ref, qo_ref, ko_ref, vo_ref):
    m = m_ref[0]
    qo_ref[0] = _row_normalize(q_ref[0]).astype(jnp.bfloat16)
    ko_ref[0] = _row_normalize(k_ref[0]).astype(jnp.bfloat16)
    vo_ref[0] = (v_ref[0] * m[:, None]).astype(jnp.bfloat16)


def _attn_kernel(q_ref, k_ref, v_ref, m_ref, o_ref, *, bq):
    iq = pl.program_id(1)
    s = jax.lax.dot_general(
        q_ref[0], k_ref[0], (((1,), (1,)), ((), ())),
        preferred_element_type=jnp.float32,
    )
    ax = jnp.minimum(jnp.abs(s), 0.99999)
    p = jnp.float32(_ACOS_PI_COEFFS[0])
    for c in _ACOS_PI_COEFFS[1:]:
        p = p * ax + jnp.float32(c)
    y = 1.0 - ax
    r = (y * jax.lax.rsqrt(y)) * p
    t = jnp.where(s >= 0, 1.0 - r, r)
    t2 = t * t
    t4 = t2 * t2
    t8 = t4 * t4
    e = (t8 * t).astype(jnp.bfloat16)
    x = jax.lax.dot_general(
        e, v_ref[0], (((1,), (0,)), ((), ())),
        preferred_element_type=jnp.float32,
    )
    mq = m_ref[0, pl.ds(iq * bq, bq)]
    o_ref[0] = _row_normalize(x * mq[:, None])


def kernel(Q, K, V, mask):
    B, H, S, D = Q.shape
    BH = B * H
    q = Q.reshape(BH, S, D)
    k = K.reshape(BH, S, D)
    v = V.reshape(BH, S, D)
    mf = mask.astype(jnp.float32)

    qn, kn, vm = pl.pallas_call(
        _prep_kernel,
        grid=(BH,),
        in_specs=[
            pl.BlockSpec((1, S, D), lambda h: (h, 0, 0)),
            pl.BlockSpec((1, S, D), lambda h: (h, 0, 0)),
            pl.BlockSpec((1, S, D), lambda h: (h, 0, 0)),
            pl.BlockSpec((1, S), lambda h: (0, 0)),
        ],
        out_specs=[
            pl.BlockSpec((1, S, D), lambda h: (h, 0, 0)),
            pl.BlockSpec((1, S, D), lambda h: (h, 0, 0)),
            pl.BlockSpec((1, S, D), lambda h: (h, 0, 0)),
        ],
        out_shape=[
            jax.ShapeDtypeStruct((BH, S, D), jnp.bfloat16),
            jax.ShapeDtypeStruct((BH, S, D), jnp.bfloat16),
            jax.ShapeDtypeStruct((BH, S, D), jnp.bfloat16),
        ],
    )(q, k, v, mf)

    bq = min(1024, S)
    grid = (BH, S // bq)

    out = pl.pallas_call(
        functools.partial(_attn_kernel, bq=bq),
        grid=grid,
        in_specs=[
            pl.BlockSpec((1, bq, D), lambda h, i: (h, i, 0)),
            pl.BlockSpec((1, S, D), lambda h, i: (h, 0, 0)),
            pl.BlockSpec((1, S, D), lambda h, i: (h, 0, 0)),
            pl.BlockSpec((1, S), lambda h, i: (0, 0)),
        ],
        out_specs=pl.BlockSpec((1, bq, D), lambda h, i: (h, i, 0)),
        out_shape=jax.ShapeDtypeStruct((BH, S, D), jnp.float32),
    )(qn, kn, vm, mf)

    return out.reshape(B, H, S, D)


# speedup vs baseline: 2.1425x; 1.0163x over previous
import functools
import math

import jax
import jax.numpy as jnp
from jax.experimental import pallas as pl

_ACOS_PI_COEFFS = (
    -0.0187293 / math.pi,
    0.0742610 / math.pi,
    -0.2121144 / math.pi,
    1.5707288 / math.pi,
)


def _row_normalize(x):
    n2 = jnp.sum(x * x, axis=-1, keepdims=True)
    return x * jax.lax.rsqrt(jnp.maximum(n2, 1e-24))


def _prep_kernel(q_ref, k_ref, v_ref, m_ref, qo_ref, ko_ref, vo_ref):
    m = m_ref[0]
    qo_ref[0] = _row_normalize(q_ref[0]).astype(jnp.bfloat16)
    ko_ref[0] = _row_normalize(k_ref[0]).astype(jnp.bfloat16)
    vo_ref[0] = (v_ref[0] * m[:, None]).astype(jnp.bfloat16)


def _attn_kernel(q_ref, k_ref, v_ref, m_ref, o_ref, *, bq):
    iq = pl.program_id(1)
    s = jax.lax.dot_general(
        q_ref[0], k_ref[0], (((1,), (1,)), ((), ())),
        preferred_element_type=jnp.float32,
    )
    ax = jnp.minimum(jnp.abs(s), 0.99999)
    p = jnp.float32(_ACOS_PI_COEFFS[0])
    for c in _ACOS_PI_COEFFS[1:]:
        p = p * ax + jnp.float32(c)
    y = 1.0 - ax
    r = (y * jax.lax.rsqrt(y)) * p
    t = jnp.where(s >= 0, 1.0 - r, r)
    t2 = t * t
    t4 = t2 * t2
    t8 = t4 * t4
    e = (t8 * t).astype(jnp.bfloat16)
    x = jax.lax.dot_general(
        e, v_ref[0], (((1,), (0,)), ((), ())),
        preferred_element_type=jnp.float32,
    )
    mq = m_ref[0, pl.ds(iq * bq, bq)]
    o_ref[0] = _row_normalize(x * mq[:, None])


def kernel(Q, K, V, mask):
    B, H, S, D = Q.shape
    BH = B * H
    q = Q.reshape(BH, S, D)
    k = K.reshape(BH, S, D)
    v = V.reshape(BH, S, D)
    mf = mask.astype(jnp.float32)

    qn, kn, vm = pl.pallas_call(
        _prep_kernel,
        grid=(BH,),
        in_specs=[
            pl.BlockSpec((1, S, D), lambda h: (h, 0, 0)),
            pl.BlockSpec((1, S, D), lambda h: (h, 0, 0)),
            pl.BlockSpec((1, S, D), lambda h: (h, 0, 0)),
            pl.BlockSpec((1, S), lambda h: (0, 0)),
        ],
        out_specs=[
            pl.BlockSpec((1, S, D), lambda h: (h, 0, 0)),
            pl.BlockSpec((1, S, D), lambda h: (h, 0, 0)),
            pl.BlockSpec((1, S, D), lambda h: (h, 0, 0)),
        ],
        out_shape=[
            jax.ShapeDtypeStruct((BH, S, D), jnp.bfloat16),
            jax.ShapeDtypeStruct((BH, S, D), jnp.bfloat16),
            jax.ShapeDtypeStruct((BH, S, D), jnp.bfloat16),
        ],
    )(q, k, v, mf)

    bq = min(2048, S)
    grid = (BH, S // bq)

    out = pl.pallas_call(
        functools.partial(_attn_kernel, bq=bq),
        grid=grid,
        in_specs=[
            pl.BlockSpec((1, bq, D), lambda h, i: (h, i, 0)),
            pl.BlockSpec((1, S, D), lambda h, i: (h, 0, 0)),
            pl.BlockSpec((1, S, D), lambda h, i: (h, 0, 0)),
            pl.BlockSpec((1, S), lambda h, i: (0, 0)),
        ],
        out_specs=pl.BlockSpec((1, bq, D), lambda h, i: (h, i, 0)),
        out_shape=jax.ShapeDtypeStruct((BH, S, D), jnp.float32),
    )(qn, kn, vm, mf)

    return out.reshape(B, H, S, D)


# speedup vs baseline: 2.2053x; 1.0293x over previous
import functools
import math

import jax
import jax.numpy as jnp
from jax.experimental import pallas as pl

_ACOS_PI_COEFFS = (
    0.01584731,
    -0.06484564,
    0.49978893,
)


def _row_normalize(x):
    n2 = jnp.sum(x * x, axis=-1, keepdims=True)
    return x * jax.lax.rsqrt(jnp.maximum(n2, 1e-24))


def _prep_kernel(q_ref, k_ref, v_ref, m_ref, qo_ref, ko_ref, vo_ref):
    m = m_ref[0]
    qo_ref[0] = _row_normalize(q_ref[0]).astype(jnp.bfloat16)
    ko_ref[0] = _row_normalize(k_ref[0]).astype(jnp.bfloat16)
    vo_ref[0] = (v_ref[0] * m[:, None]).astype(jnp.bfloat16)


def _attn_kernel(q_ref, k_ref, v_ref, m_ref, o_ref, *, bq):
    iq = pl.program_id(1)
    s = jax.lax.dot_general(
        q_ref[0], k_ref[0], (((1,), (1,)), ((), ())),
        preferred_element_type=jnp.float32,
    )
    ax = jnp.minimum(jnp.abs(s), 0.99999)
    p = jnp.float32(_ACOS_PI_COEFFS[0])
    for c in _ACOS_PI_COEFFS[1:]:
        p = p * ax + jnp.float32(c)
    y = 1.0 - ax
    r = (y * jax.lax.rsqrt(y)) * p
    t = jnp.where(s >= 0, 1.0 - r, r)
    e = jnp.exp2(9.0 * jnp.log2(t)).astype(jnp.bfloat16)
    x = jax.lax.dot_general(
        e, v_ref[0], (((1,), (0,)), ((), ())),
        preferred_element_type=jnp.float32,
    )
    mq = m_ref[0, pl.ds(iq * bq, bq)]
    o_ref[0] = _row_normalize(x * mq[:, None])


def kernel(Q, K, V, mask):
    B, H, S, D = Q.shape
    BH = B * H
    q = Q.reshape(BH, S, D)
    k = K.reshape(BH, S, D)
    v = V.reshape(BH, S, D)
    mf = mask.astype(jnp.float32)

    qn, kn, vm = pl.pallas_call(
        _prep_kernel,
        grid=(BH,),
        in_specs=[
            pl.BlockSpec((1, S, D), lambda h: (h, 0, 0)),
            pl.BlockSpec((1, S, D), lambda h: (h, 0, 0)),
            pl.BlockSpec((1, S, D), lambda h: (h, 0, 0)),
            pl.BlockSpec((1, S), lambda h: (0, 0)),
        ],
        out_specs=[
            pl.BlockSpec((1, S, D), lambda h: (h, 0, 0)),
            pl.BlockSpec((1, S, D), lambda h: (h, 0, 0)),
            pl.BlockSpec((1, S, D), lambda h: (h, 0, 0)),
        ],
        out_shape=[
            jax.ShapeDtypeStruct((BH, S, D), jnp.bfloat16),
            jax.ShapeDtypeStruct((BH, S, D), jnp.bfloat16),
            jax.ShapeDtypeStruct((BH, S, D), jnp.bfloat16),
        ],
    )(q, k, v, mf)

    bq = min(2048, S)
    grid = (BH, S // bq)

    out = pl.pallas_call(
        functools.partial(_attn_kernel, bq=bq),
        grid=grid,
        in_specs=[
            pl.BlockSpec((1, bq, D), lambda h, i: (h, i, 0)),
            pl.BlockSpec((1, S, D), lambda h, i: (h, 0, 0)),
            pl.BlockSpec((1, S, D), lambda h, i: (h, 0, 0)),
            pl.BlockSpec((1, S), lambda h, i: (0, 0)),
        ],
        out_specs=pl.BlockSpec((1, bq, D), lambda h, i: (h, i, 0)),
        out_shape=jax.ShapeDtypeStruct((BH, S, D), jnp.float32),
    )(qn, kn, vm, mf)

    return out.reshape(B, H, S, D)


# speedup vs baseline: 2.2455x; 1.0182x over previous
import functools
import math

import jax
import jax.numpy as jnp
from jax.experimental import pallas as pl

_G_COEFFS = (
    0.028285502 / math.pi,
    0.00090954325 / math.pi,
    0.12907177 / math.pi,
    1.4126761 / math.pi,
)


def _row_normalize(x):
    n2 = jnp.sum(x * x, axis=-1, keepdims=True)
    return x * jax.lax.rsqrt(jnp.maximum(n2, 1e-24))


def _prep_kernel(q_ref, k_ref, v_ref, m_ref, qo_ref, ko_ref, vo_ref):
    m = m_ref[0]
    qo_ref[0] = _row_normalize(q_ref[0]).astype(jnp.bfloat16)
    ko_ref[0] = _row_normalize(k_ref[0]).astype(jnp.bfloat16)
    vo_ref[0] = (v_ref[0] * m[:, None]).astype(jnp.bfloat16)


def _attn_kernel(q_ref, k_ref, v_ref, m_ref, o_ref, *, bq):
    iq = pl.program_id(1)
    s = jax.lax.dot_general(
        q_ref[0], k_ref[0], (((1,), (1,)), ((), ())),
        preferred_element_type=jnp.float32,
    )
    y = jnp.maximum(1.0 - s, 1e-5)
    p = jnp.float32(_G_COEFFS[0])
    for c in _G_COEFFS[1:]:
        p = p * y + jnp.float32(c)
    t = 1.0 - (y * jax.lax.rsqrt(y)) * p
    t = jnp.maximum(t, 1e-30)
    e = jnp.exp2(9.0 * jnp.log2(t)).astype(jnp.bfloat16)
    x = jax.lax.dot_general(
        e, v_ref[0], (((1,), (0,)), ((), ())),
        preferred_element_type=jnp.float32,
    )
    mq = m_ref[0, pl.ds(iq * bq, bq)]
    o_ref[0] = _row_normalize(x * mq[:, None])


def kernel(Q, K, V, mask):
    B, H, S, D = Q.shape
    BH = B * H
    q = Q.reshape(BH, S, D)
    k = K.reshape(BH, S, D)
    v = V.reshape(BH, S, D)
    mf = mask.astype(jnp.float32)

    qn, kn, vm = pl.pallas_call(
        _prep_kernel,
        grid=(BH,),
        in_specs=[
            pl.BlockSpec((1, S, D), lambda h: (h, 0, 0)),
            pl.BlockSpec((1, S, D), lambda h: (h, 0, 0)),
            pl.BlockSpec((1, S, D), lambda h: (h, 0, 0)),
            pl.BlockSpec((1, S), lambda h: (0, 0)),
        ],
        out_specs=[
            pl.BlockSpec((1, S, D), lambda h: (h, 0, 0)),
            pl.BlockSpec((1, S, D), lambda h: (h, 0, 0)),
            pl.BlockSpec((1, S, D), lambda h: (h, 0, 0)),
        ],
        out_shape=[
            jax.ShapeDtypeStruct((BH, S, D), jnp.bfloat16),
            jax.ShapeDtypeStruct((BH, S, D), jnp.bfloat16),
            jax.ShapeDtypeStruct((BH, S, D), jnp.bfloat16),
        ],
    )(q, k, v, mf)

    bq = min(2048, S)
    grid = (BH, S // bq)

    out = pl.pallas_call(
        functools.partial(_attn_kernel, bq=bq),
        grid=grid,
        in_specs=[
            pl.BlockSpec((1, bq, D), lambda h, i: (h, i, 0)),
            pl.BlockSpec((1, S, D), lambda h, i: (h, 0, 0)),
            pl.BlockSpec((1, S, D), lambda h, i: (h, 0, 0)),
            pl.BlockSpec((1, S), lambda h, i: (0, 0)),
        ],
        out_specs=pl.BlockSpec((1, bq, D), lambda h, i: (h, i, 0)),
        out_shape=jax.ShapeDtypeStruct((BH, S, D), jnp.float32),
    )(qn, kn, vm, mf)

    return out.reshape(B, H, S, D)
